# baseline (speedup 1.0000x reference)
import functools

import jax
import jax.numpy as jnp
from jax import lax
from jax.experimental import pallas as pl
from jax.experimental.pallas import tpu as pltpu
from jax.experimental.pallas import tpu_sc as plsc

VOCAB = 1000000
DIM = 64
SEQ = 200
BATCH = 4096

NC = 2
NS = 16
NW = NC * NS
L = 16

B = SEQ * BATCH
PER_W = B // NW
GCH = 128
NCHG = PER_W // GCH
NBUF = 2


def _emb_kernel(table_hbm, idx_hbm, par_hbm, out_hbm,
                idx_v, par_c, rows_v, gsem, psem, osem):
    wid = lax.axis_index("s") * NC + lax.axis_index("c")
    base = wid * PER_W

    pltpu.sync_copy(idx_hbm.at[wid], idx_v)

    pltpu.async_copy(table_hbm.at[idx_v.at[0]], rows_v.at[0], gsem)
    pltpu.async_copy(par_hbm.at[wid, 0], par_c.at[0], psem)

    lanes = lax.iota(jnp.int32, L)

    def body(jg, _):
        for bb in range(NBUF):
            j = jg * NBUF + bb
            pltpu.make_async_copy(
                table_hbm.at[idx_v.at[0]], rows_v.at[bb], gsem
            ).wait()
            pltpu.make_async_copy(
                par_hbm.at[wid, 0], par_c.at[bb], psem
            ).wait()

            for g in range(GCH // L):
                rows16 = lanes + (g * L)
                p16 = par_c[bb, pl.ds(g * L, L)]
                src0 = p16 * DIM
                for c in range(DIM):
                    v = plsc.load_gather(
                        rows_v.at[bb], [rows16, src0 + c]
                    )
                    plsc.store_scatter(
                        rows_v.at[bb], [rows16, lanes * 0 + c], v
                    )

            pltpu.async_copy(
                rows_v.at[bb], out_hbm.at[pl.ds(base + j * GCH, GCH)],
                osem,
            )

            @pl.when(j >= 1)
            def _():
                pltpu.make_async_copy(
                    rows_v.at[bb], out_hbm.at[pl.ds(base, GCH)], osem
                ).wait()

            @pl.when(j + 1 < NCHG)
            def _():
                nb = (bb + 1) % NBUF
                pltpu.async_copy(
                    table_hbm.at[idx_v.at[j + 1]], rows_v.at[nb], gsem
                )
                pltpu.async_copy(
                    par_hbm.at[wid, j + 1], par_c.at[nb], psem
                )

        return 0

    lax.fori_loop(0, NCHG // NBUF, body, 0)

    pltpu.make_async_copy(
        rows_v.at[0], out_hbm.at[pl.ds(base, GCH)], osem
    ).wait()


@jax.jit
def _emb(table2, idx3, par3):
    mesh = plsc.VectorSubcoreMesh(
        core_axis_name="c", subcore_axis_name="s",
        num_cores=NC, num_subcores=NS,
    )
    run = pl.kernel(
        _emb_kernel,
        out_type=jax.ShapeDtypeStruct((B, 2 * DIM), jnp.float32),
        mesh=mesh,
        scratch_types=[
            pltpu.VMEM((NCHG, GCH), jnp.int32),
            pltpu.VMEM((NBUF, GCH), jnp.int32),
            pltpu.VMEM((NBUF, GCH, 2 * DIM), jnp.float32),
            pltpu.SemaphoreType.DMA,
            pltpu.SemaphoreType.DMA,
            pltpu.SemaphoreType.DMA,
        ],
        compiler_params=pltpu.CompilerParams(
            use_tc_tiling_on_sc=True,
            needs_layout_passes=False,
        ),
    )
    return run(table2, idx3, par3)


def kernel(src_input, table):
    idx = src_input.reshape(B).astype(jnp.int32)
    idx3 = (idx // 2).reshape(NW, NCHG, GCH)
    par3 = (idx & 1).reshape(NW, NCHG, GCH)
    table2 = table.reshape(VOCAB // 2, 2 * DIM)
    out = _emb(table2, idx3, par3)
    emb = out[:, :DIM]
    return emb.reshape(SEQ, BATCH, DIM)


# speedup vs baseline: 2.7486x; 2.7486x over previous
import functools

import jax
import jax.numpy as jnp
from jax import lax
from jax.experimental import pallas as pl
from jax.experimental.pallas import tpu as pltpu
from jax.experimental.pallas import tpu_sc as plsc

VOCAB = 1000000
DIM = 64
SEQ = 200
BATCH = 4096

NC = 2
NS = 16
NW = NC * NS

B = SEQ * BATCH
PER_W = B // NW
GCH = 128
NCHG = PER_W // GCH
NBUF = 4
KAHEAD = 2


def _emb_kernel(table_hbm, idx_hbm, out_hbm, idx_v, rows_v, gsem, osem):
    wid = lax.axis_index("s") * NC + lax.axis_index("c")
    base2 = wid * PER_W * 2

    pltpu.sync_copy(idx_hbm.at[wid], idx_v)

    for g in range(KAHEAD):
        pltpu.async_copy(table_hbm.at[idx_v.at[g]], rows_v.at[g], gsem)

    def body(j, _):
        b = lax.rem(j, NBUF)
        pltpu.make_async_copy(
            table_hbm.at[idx_v.at[0]], rows_v.at[b], gsem
        ).wait()
        pltpu.async_copy(
            rows_v.at[b], out_hbm.at[pl.ds(base2 + j * 2 * GCH, 2 * GCH)],
            osem,
        )

        @pl.when(j >= NBUF - KAHEAD)
        def _():
            pltpu.make_async_copy(
                rows_v.at[b], out_hbm.at[pl.ds(base2, 2 * GCH)], osem
            ).wait()

        @pl.when(j + KAHEAD < NCHG)
        def _():
            bn = lax.rem(j + KAHEAD, NBUF)
            pltpu.async_copy(
                table_hbm.at[idx_v.at[j + KAHEAD]], rows_v.at[bn], gsem
            )

        return 0

    lax.fori_loop(0, NCHG, body, 0)

    for _ in range(NBUF - KAHEAD):
        pltpu.make_async_copy(
            rows_v.at[0], out_hbm.at[pl.ds(base2, 2 * GCH)], osem
        ).wait()


@jax.jit
def _emb(table, idx3):
    mesh = plsc.VectorSubcoreMesh(
        core_axis_name="c", subcore_axis_name="s",
        num_cores=NC, num_subcores=NS,
    )
    run = pl.kernel(
        _emb_kernel,
        out_type=jax.ShapeDtypeStruct((2 * B, DIM), jnp.float32),
        mesh=mesh,
        scratch_types=[
            pltpu.VMEM((NCHG, 2 * GCH), jnp.int32),
            pltpu.VMEM((NBUF, 2 * GCH, DIM), jnp.float32),
            pltpu.SemaphoreType.DMA,
            pltpu.SemaphoreType.DMA,
        ],
        compiler_params=pltpu.CompilerParams(use_tc_tiling_on_sc=False),
    )
    return run(table, idx3)


def kernel(src_input, table):
    idx = src_input.reshape(B).astype(jnp.int32)
    idxr = jnp.repeat(idx, 2)
    idx3 = idxr.reshape(NW, NCHG, 2 * GCH)
    out = _emb(table, idx3)
    emb = out.reshape(B, 2 * DIM)[:, :DIM]
    return emb.reshape(SEQ, BATCH, DIM)
